# baseline (speedup 1.0000x reference)
import jax
import jax.numpy as jnp
from jax.experimental import pallas as pl
from jax.experimental.pallas import tpu as pltpu

_B, _K = 16384, 64
_NCH = 16
_CB = _B // _NCH


def _body(u_hbm, v_hbm, out_ref, u_v, v_v, sems):
    copies = []
    for c in range(_NCH):
        cu = pltpu.make_async_copy(
            u_hbm.at[:, pl.ds(c * _CB, _CB)],
            u_v.at[:, pl.ds(c * _CB, _CB)],
            sems.at[0, c],
        )
        cv = pltpu.make_async_copy(
            v_hbm.at[:, pl.ds(c * _CB, _CB)],
            v_v.at[:, pl.ds(c * _CB, _CB)],
            sems.at[1, c],
        )
        cu.start()
        cv.start()
        copies.append((cu, cv))
    for c in range(_NCH):
        cu, cv = copies[c]
        cu.wait()
        cv.wait()
        s = jnp.sum(
            u_v[:, pl.ds(c * _CB, _CB)] * v_v[:, pl.ds(c * _CB, _CB)], axis=0
        )
        out_ref[pl.ds(c * (_CB // 128), _CB // 128), :] = s.reshape(_CB // 128, 128)


def kernel(gu, gi):
    out = pl.pallas_call(
        _body,
        in_specs=[
            pl.BlockSpec(memory_space=pltpu.HBM),
            pl.BlockSpec(memory_space=pltpu.HBM),
        ],
        out_specs=pl.BlockSpec(memory_space=pltpu.VMEM),
        out_shape=jax.ShapeDtypeStruct((_B // 128, 128), jnp.float32),
        scratch_shapes=[
            pltpu.VMEM((_K, _B), jnp.float32),
            pltpu.VMEM((_K, _B), jnp.float32),
            pltpu.SemaphoreType.DMA((2, _NCH)),
        ],
    )(gu.T, gi.T)
    return out.reshape(_B)


# speedup vs baseline: 1.0171x; 1.0171x over previous
import jax
import jax.numpy as jnp
from jax.experimental import pallas as pl
from jax.experimental.pallas import tpu as pltpu

_B, _K = 16384, 64
_NCH = 4
_CB = _B // _NCH


def _body(u_hbm, v_hbm, out_ref, u_v, v_v, sems):
    copies = []
    for c in range(_NCH):
        cu = pltpu.make_async_copy(
            u_hbm.at[:, pl.ds(c * _CB, _CB)],
            u_v.at[:, pl.ds(c * _CB, _CB)],
            sems.at[0, c],
        )
        cv = pltpu.make_async_copy(
            v_hbm.at[:, pl.ds(c * _CB, _CB)],
            v_v.at[:, pl.ds(c * _CB, _CB)],
            sems.at[1, c],
        )
        cu.start()
        cv.start()
        copies.append((cu, cv))
    for c in range(_NCH):
        cu, cv = copies[c]
        cu.wait()
        cv.wait()
        s = jnp.sum(
            u_v[:, pl.ds(c * _CB, _CB)] * v_v[:, pl.ds(c * _CB, _CB)], axis=0
        )
        out_ref[pl.ds(c * (_CB // 128), _CB // 128), :] = s.reshape(_CB // 128, 128)


def kernel(gu, gi):
    out = pl.pallas_call(
        _body,
        in_specs=[
            pl.BlockSpec(memory_space=pltpu.HBM),
            pl.BlockSpec(memory_space=pltpu.HBM),
        ],
        out_specs=pl.BlockSpec(memory_space=pltpu.VMEM),
        out_shape=jax.ShapeDtypeStruct((_B // 128, 128), jnp.float32),
        scratch_shapes=[
            pltpu.VMEM((_K, _B), jnp.float32),
            pltpu.VMEM((_K, _B), jnp.float32),
            pltpu.SemaphoreType.DMA((2, _NCH)),
        ],
    )(gu.T, gi.T)
    return out.reshape(_B)
